# Initial kernel scaffold; baseline (speedup 1.0000x reference)
import jax
import jax.numpy as jnp
from jax.experimental import pallas as pl


def kernel(xb, W_raw, b_raw, W_left, b_left, W_sup, b_sup, W_right, b_right):
    raise NotImplementedError("write your pallas kernel here")


# speedup vs baseline: 3.9171x; 3.9171x over previous
import jax
import jax.numpy as jnp
from jax.experimental import pallas as pl

_B, _D = 32768, 1024
_TB = 2048
_LANES = 128


def _fused_body(x_ref, w_ref, b_ref, o_ref):
    logits = jnp.dot(x_ref[...], w_ref[...],
                     preferred_element_type=jnp.float32) + b_ref[...]
    r0 = logits[:, 0]
    r1 = logits[:, 1]
    r2 = logits[:, 2]
    e0 = (r0 >= r1) & (r0 >= r2)
    e1 = (~e0) & (r1 >= r2)

    def arg3(c0, c1, c2):
        a0 = (c0 >= c1) & (c0 >= c2)
        a1 = (~a0) & (c1 >= c2)
        return jnp.where(a0, 0, jnp.where(a1, 1, 2)).astype(jnp.int32)

    a_left = arg3(logits[:, 3], logits[:, 4], logits[:, 5])
    a_sup = arg3(logits[:, 6], logits[:, 7], logits[:, 8])
    a_right = arg3(logits[:, 9], logits[:, 10], logits[:, 11])

    final_arg = jnp.where(e0, a_left, jnp.where(e1, a_sup, a_right))
    relabeled = jnp.where(e0, 1, jnp.where(e1, 0, 2)).astype(jnp.int32)
    o_ref[...] = relabeled * 3 + final_arg + 1


def kernel(xb, W_raw, b_raw, W_left, b_left, W_sup, b_sup, W_right, b_right):
    xb = xb.astype(jnp.float32)
    W = jnp.concatenate([W_raw, W_left, W_sup, W_right], axis=1)
    Wp = jnp.pad(W, ((0, 0), (0, _LANES - 12)))
    b = jnp.concatenate([b_raw, b_left, b_sup, b_right])
    bp = jnp.pad(b, (0, _LANES - 12)).reshape(1, _LANES)

    nb = _B // _TB
    out = pl.pallas_call(
        _fused_body,
        grid=(nb,),
        in_specs=[
            pl.BlockSpec((_TB, _D), lambda i: (i, 0)),
            pl.BlockSpec((_D, _LANES), lambda i: (0, 0)),
            pl.BlockSpec((1, _LANES), lambda i: (0, 0)),
        ],
        out_specs=pl.BlockSpec((_TB,), lambda i: (i,)),
        out_shape=jax.ShapeDtypeStruct((_B,), jnp.int32),
    )(xb, Wp, bp)
    return out


# speedup vs baseline: 5.2334x; 1.3360x over previous
import jax
import jax.numpy as jnp
from jax.experimental import pallas as pl

_B, _D = 32768, 1024
_TB = 2048
_LANES = 128


def _fused_body(x_ref, w_ref, b_ref, o_ref):
    logits = jnp.dot(x_ref[...], w_ref[...],
                     preferred_element_type=jnp.float32)
    lt = logits.T + b_ref[...]

    r0 = lt[0:1, :]
    r1 = lt[1:2, :]
    r2 = lt[2:3, :]
    e0 = (r0 >= r1) & (r0 >= r2)
    e1 = (~e0) & (r1 >= r2)

    def arg3(c0, c1, c2):
        a0 = (c0 >= c1) & (c0 >= c2)
        a1 = (~a0) & (c1 >= c2)
        return jnp.where(a0, 0, jnp.where(a1, 1, 2)).astype(jnp.int32)

    a_left = arg3(lt[3:4, :], lt[4:5, :], lt[5:6, :])
    a_sup = arg3(lt[6:7, :], lt[7:8, :], lt[8:9, :])
    a_right = arg3(lt[9:10, :], lt[10:11, :], lt[11:12, :])

    final_arg = jnp.where(e0, a_left, jnp.where(e1, a_sup, a_right))
    relabeled = jnp.where(e0, 1, jnp.where(e1, 0, 2)).astype(jnp.int32)
    o_ref[...] = (relabeled * 3 + final_arg + 1).reshape(1, 1, _TB)


def kernel(xb, W_raw, b_raw, W_left, b_left, W_sup, b_sup, W_right, b_right):
    xb = xb.astype(jnp.float32)
    W = jnp.concatenate([W_raw, W_left, W_sup, W_right], axis=1)
    Wp = jnp.pad(W, ((0, 0), (0, _LANES - 12)))
    b = jnp.concatenate([b_raw, b_left, b_sup, b_right])
    bp = jnp.pad(b, (0, _LANES - 12)).reshape(_LANES, 1)

    nb = _B // _TB
    out = pl.pallas_call(
        _fused_body,
        grid=(nb,),
        in_specs=[
            pl.BlockSpec((_TB, _D), lambda i: (i, 0)),
            pl.BlockSpec((_D, _LANES), lambda i: (0, 0)),
            pl.BlockSpec((_LANES, 1), lambda i: (0, 0)),
        ],
        out_specs=pl.BlockSpec((1, 1, _TB), lambda i: (i, 0, 0)),
        out_shape=jax.ShapeDtypeStruct((nb, 1, _TB), jnp.int32),
    )(xb, Wp, bp)
    return out.reshape(_B)
